# baseline (device time: 36560 ns/iter reference)
import jax
import jax.numpy as jnp
from jax import lax
from jax.experimental import pallas as pl
from jax.experimental.pallas import tpu as pltpu

N_DEV = 8
B, S, H, Dh, Dr = 2, 256, 16, 64, 32
D = 1024
DC = 64
BS = B * S
HPD = H // N_DEV
HB = HPD * Dh
RB = HPD * Dr
SCALE = (Dh + Dr) ** -0.5
BF = jnp.bfloat16
F32 = jnp.float32

N_BUF = 5
N_PEER = N_DEV - 1


def kernel(x, Wdkv, Wuk, Wuv, Wq, Wqr, Wkr, Wo):
    me_out = lax.axis_index("i")
    x2 = x.reshape(BS, D)
    wqr_m = lax.dynamic_slice_in_dim(Wqr, me_out * RB, RB, 1)

    def body(x_ref, wdkv_ref, wuk_ref, wuv_ref, wq_ref, wqr_ref, wkr_ref,
             wo_ref, out_ref,
             c_gat, uk_send, uv_send, uk_gat, uv_gat,
             kacc, vacc, kb_buf, vb_buf, q_buf, qr_buf, kr_buf,
             o_loc, o_all, wq_v, wo_v, wo_bf,
             send_sems, recv_sems, loc_sems):
        me = lax.axis_index("i")

        wq_cp = pltpu.make_async_copy(
            wq_ref.at[:, pl.ds(me * HB, HB)], wq_v, loc_sems.at[0])
        wo_cp = pltpu.make_async_copy(wo_ref, wo_v, loc_sems.at[1])
        wq_cp.start()
        wo_cp.start()

        barrier_sem = pltpu.get_barrier_semaphore()
        for k in range(1, N_DEV):
            pl.semaphore_signal(
                barrier_sem, inc=1,
                device_id=(lax.rem(me + k, N_DEV),),
                device_id_type=pl.DeviceIdType.MESH,
            )
        pl.semaphore_wait(barrier_sem, N_PEER)

        xb = x_ref[...].astype(BF)

        c_gat[0] = jnp.dot(xb, wdkv_ref[...].astype(BF),
                           preferred_element_type=F32).astype(BF)
        for d in range(N_DEV):
            uk_send[d] = wuk_ref[:, d * HB:(d + 1) * HB].astype(BF)
            uv_send[d] = wuv_ref[:, d * HB:(d + 1) * HB].astype(BF)

        sends = []

        def push(src, dst, bi, k, dest):
            r = pltpu.make_async_remote_copy(
                src_ref=src,
                dst_ref=dst,
                send_sem=send_sems.at[bi, k - 1],
                recv_sem=recv_sems.at[bi, N_DEV - k - 1],
                device_id=(dest,),
                device_id_type=pl.DeviceIdType.MESH,
            )
            r.start()
            sends.append(r)

        for k in range(1, N_DEV):
            dest = lax.rem(me + k, N_DEV)
            slot = N_DEV - k
            push(c_gat.at[0], c_gat.at[slot], 0, k, dest)
            push(uk_send.at[dest], uk_gat.at[slot], 1, k, dest)
            push(uv_send.at[dest], uv_gat.at[slot], 2, k, dest)

        wq_cp.wait()
        q_buf[...] = jnp.dot(xb, wq_v[...].astype(BF),
                             preferred_element_type=F32).astype(BF)
        qr_buf[...] = jnp.dot(xb, wqr_ref[...].astype(BF),
                              preferred_element_type=F32).astype(BF)
        kr_buf[...] = jnp.dot(xb, wkr_ref[...].astype(BF),
                              preferred_element_type=F32).astype(BF)
        kacc[...] = jnp.dot(c_gat[0], uk_send[me],
                            preferred_element_type=F32)
        vacc[...] = jnp.dot(c_gat[0], uv_send[me],
                            preferred_element_type=F32)
        wo_cp.wait()
        wo_bf[...] = wo_v[...].astype(BF)

        def wait_recv(bi, s, dst):
            recv = pltpu.make_async_remote_copy(
                src_ref=dst,
                dst_ref=dst,
                send_sem=send_sems.at[bi, s - 1],
                recv_sem=recv_sems.at[bi, s - 1],
                device_id=(me,),
                device_id_type=pl.DeviceIdType.MESH,
            )
            recv.wait_recv()

        for s in range(1, N_DEV):
            wait_recv(0, s, c_gat.at[s])
            wait_recv(1, s, uk_gat.at[s])
            wait_recv(2, s, uv_gat.at[s])
            kacc[...] = kacc[...] + jnp.dot(c_gat[s], uk_gat[s],
                                            preferred_element_type=F32)
            vacc[...] = vacc[...] + jnp.dot(c_gat[s], uv_gat[s],
                                            preferred_element_type=F32)

        kb_buf[...] = kacc[...].astype(BF)
        vb_buf[...] = vacc[...].astype(BF)

        for b in range(B):
            rows = slice(b * S, (b + 1) * S)
            kr = kr_buf[rows, :]
            for hh in range(HPD):
                dcols = slice(hh * Dh, (hh + 1) * Dh)
                rcols = slice(hh * Dr, (hh + 1) * Dr)
                q = q_buf[rows, dcols]
                k = kb_buf[rows, dcols]
                qr = qr_buf[rows, rcols]
                v = vb_buf[rows, dcols]
                s_nope = lax.dot_general(
                    q, k, (((1,), (1,)), ((), ())),
                    preferred_element_type=F32)
                s_rope = lax.dot_general(
                    qr, kr, (((1,), (1,)), ((), ())),
                    preferred_element_type=F32)
                sc = (s_nope + s_rope) * SCALE
                m = jnp.max(sc, axis=1, keepdims=True)
                e = jnp.exp(sc - m)
                p = (e / jnp.sum(e, axis=1, keepdims=True)).astype(BF)
                o = jnp.dot(p, v, preferred_element_type=F32)
                o_loc[rows, dcols] = o.astype(BF)
            for k2 in range(1, N_DEV):
                dest = lax.rem(me + k2, N_DEV)
                push(o_loc.at[pl.ds(b * S, S), :],
                     o_all.at[pl.ds(b * S, S), pl.ds(me * HB, HB)],
                     3 + b, k2, dest)

        oloc_cp = pltpu.make_async_copy(
            o_loc, o_all.at[:, pl.ds(me * HB, HB)], loc_sems.at[2])
        oloc_cp.start()

        for s in range(1, N_DEV):
            wait_recv(3, s, o_all.at[0:S, 0:HB])
            wait_recv(4, s, o_all.at[0:S, 0:HB])
        oloc_cp.wait()

        out = jnp.dot(o_all[...], wo_bf[...], preferred_element_type=F32)
        for b in range(B):
            out_ref[b] = out[b * S:(b + 1) * S, :]

        for r in sends:
            r.wait_send()

    vmem = pl.BlockSpec(memory_space=pltpu.VMEM)
    hbm = pl.BlockSpec(memory_space=pl.ANY)
    out3 = pl.pallas_call(
        body,
        out_shape=jax.ShapeDtypeStruct((B, S, D), F32),
        in_specs=[vmem, vmem, vmem, vmem, hbm, vmem, vmem, hbm],
        out_specs=vmem,
        scratch_shapes=[
            pltpu.VMEM((N_DEV, BS, DC), BF),
            pltpu.VMEM((N_DEV, DC, HB), BF),
            pltpu.VMEM((N_DEV, DC, HB), BF),
            pltpu.VMEM((N_DEV, DC, HB), BF),
            pltpu.VMEM((N_DEV, DC, HB), BF),
            pltpu.VMEM((BS, HB), F32),
            pltpu.VMEM((BS, HB), F32),
            pltpu.VMEM((BS, HB), BF),
            pltpu.VMEM((BS, HB), BF),
            pltpu.VMEM((BS, HB), BF),
            pltpu.VMEM((BS, RB), BF),
            pltpu.VMEM((BS, Dr), BF),
            pltpu.VMEM((BS, HB), BF),
            pltpu.VMEM((BS, D), BF),
            pltpu.VMEM((D, HB), F32),
            pltpu.VMEM((D, D), F32),
            pltpu.VMEM((D, D), BF),
            pltpu.SemaphoreType.DMA((N_BUF, N_PEER)),
            pltpu.SemaphoreType.DMA((N_BUF, N_PEER)),
            pltpu.SemaphoreType.DMA((3,)),
        ],
        compiler_params=pltpu.CompilerParams(collective_id=0),
    )(x2, Wdkv, Wuk, Wuv, Wq, wqr_m, Wkr, Wo)
    return out3


# device time: 32636 ns/iter; 1.1202x vs baseline; 1.1202x over previous
import jax
import jax.numpy as jnp
from jax import lax
from jax.experimental import pallas as pl
from jax.experimental.pallas import tpu as pltpu

N_DEV = 8
B, S, H, Dh, Dr = 2, 256, 16, 64, 32
D = 1024
DC = 64
BS = B * S
HPD = H // N_DEV
HB = HPD * Dh
RB = HPD * Dr
SCALE = (Dh + Dr) ** -0.5
BF = jnp.bfloat16
F32 = jnp.float32

N_BUF = 4
N_PEER = N_DEV - 1


def kernel(x, Wdkv, Wuk, Wuv, Wq, Wqr, Wkr, Wo):
    me_out = lax.axis_index("i")
    x2 = x.reshape(BS, D)
    wq_m = lax.dynamic_slice_in_dim(Wq, me_out * HB, HB, 1)
    wqr_m = lax.dynamic_slice_in_dim(Wqr, me_out * RB, RB, 1)

    def body(x_ref, wdkv_ref, wuk_ref, wuv_ref, wq_ref, wqr_ref, wkr_ref,
             wo_ref, out_ref,
             c_gat, kv_send, kv_gat, kvacc, kb_buf, vb_buf,
             q_buf, qr_buf, kr_buf, o_loc, o_all, wo_bf,
             send_sems, recv_sems, loc_sems):
        me = lax.axis_index("i")

        barrier_sem = pltpu.get_barrier_semaphore()
        for k in range(1, N_DEV):
            pl.semaphore_signal(
                barrier_sem, inc=1,
                device_id=(lax.rem(me + k, N_DEV),),
                device_id_type=pl.DeviceIdType.MESH,
            )
        pl.semaphore_wait(barrier_sem, N_PEER)

        xb = x_ref[...].astype(BF)

        c_gat[0] = jnp.dot(xb, wdkv_ref[...].astype(BF),
                           preferred_element_type=F32).astype(BF)
        for d in range(N_DEV):
            kv_send[d, :, 0:HB] = wuk_ref[:, d * HB:(d + 1) * HB].astype(BF)
            kv_send[d, :, HB:2 * HB] = (
                wuv_ref[:, d * HB:(d + 1) * HB].astype(BF))

        sends = []

        def push(src, dst, bi, k, dest):
            r = pltpu.make_async_remote_copy(
                src_ref=src,
                dst_ref=dst,
                send_sem=send_sems.at[bi, k - 1],
                recv_sem=recv_sems.at[bi, N_DEV - k - 1],
                device_id=(dest,),
                device_id_type=pl.DeviceIdType.MESH,
            )
            r.start()
            sends.append(r)

        for k in range(1, N_DEV):
            dest = lax.rem(me + k, N_DEV)
            slot = N_DEV - k
            push(c_gat.at[0], c_gat.at[slot], 0, k, dest)
            push(kv_send.at[dest], kv_gat.at[slot], 1, k, dest)

        q_buf[...] = jnp.dot(xb, wq_ref[...].astype(BF),
                             preferred_element_type=F32).astype(BF)
        qr_buf[...] = jnp.dot(xb, wqr_ref[...].astype(BF),
                              preferred_element_type=F32).astype(BF)
        kr_buf[...] = jnp.dot(xb, wkr_ref[...].astype(BF),
                              preferred_element_type=F32).astype(BF)
        kvacc[...] = jnp.dot(c_gat[0], kv_send[me],
                             preferred_element_type=F32)
        wo_bf[...] = wo_ref[...].astype(BF)

        def wait_recv(bi, s, dst):
            recv = pltpu.make_async_remote_copy(
                src_ref=dst,
                dst_ref=dst,
                send_sem=send_sems.at[bi, s - 1],
                recv_sem=recv_sems.at[bi, s - 1],
                device_id=(me,),
                device_id_type=pl.DeviceIdType.MESH,
            )
            recv.wait_recv()

        for s in range(1, N_DEV):
            wait_recv(0, s, c_gat.at[s])
            wait_recv(1, s, kv_gat.at[s])
            kvacc[...] = kvacc[...] + jnp.dot(c_gat[s], kv_gat[s],
                                              preferred_element_type=F32)

        kb_buf[...] = kvacc[:, 0:HB].astype(BF)
        vb_buf[...] = kvacc[:, HB:2 * HB].astype(BF)

        for b in range(B):
            rows = slice(b * S, (b + 1) * S)
            kr = kr_buf[rows, :]
            for hh in range(HPD):
                dcols = slice(hh * Dh, (hh + 1) * Dh)
                rcols = slice(hh * Dr, (hh + 1) * Dr)
                q = q_buf[rows, dcols]
                k = kb_buf[rows, dcols]
                qr = qr_buf[rows, rcols]
                v = vb_buf[rows, dcols]
                s_nope = lax.dot_general(
                    q, k, (((1,), (1,)), ((), ())),
                    preferred_element_type=F32)
                s_rope = lax.dot_general(
                    qr, kr, (((1,), (1,)), ((), ())),
                    preferred_element_type=F32)
                sc = (s_nope + s_rope) * SCALE
                m = jnp.max(sc, axis=1, keepdims=True)
                e = jnp.exp(sc - m)
                p = (e / jnp.sum(e, axis=1, keepdims=True)).astype(BF)
                o = jnp.dot(p, v, preferred_element_type=F32)
                o_loc[rows, dcols] = o.astype(BF)
            for k2 in range(1, N_DEV):
                dest = lax.rem(me + k2, N_DEV)
                push(o_loc.at[pl.ds(b * S, S), :],
                     o_all.at[pl.ds(b * S, S), pl.ds(me * HB, HB)],
                     2 + b, k2, dest)

        oloc_cp = pltpu.make_async_copy(
            o_loc, o_all.at[:, pl.ds(me * HB, HB)], loc_sems.at[0])
        oloc_cp.start()

        for s in range(1, N_DEV):
            wait_recv(2, s, o_all.at[0:S, 0:HB])
            wait_recv(3, s, o_all.at[0:S, 0:HB])
        oloc_cp.wait()

        out = jnp.dot(o_all[...], wo_bf[...], preferred_element_type=F32)
        for b in range(B):
            out_ref[b] = out[b * S:(b + 1) * S, :]

        for r in sends:
            r.wait_send()

    vmem = pl.BlockSpec(memory_space=pltpu.VMEM)
    out3 = pl.pallas_call(
        body,
        out_shape=jax.ShapeDtypeStruct((B, S, D), F32),
        in_specs=[vmem] * 8,
        out_specs=vmem,
        scratch_shapes=[
            pltpu.VMEM((N_DEV, BS, DC), BF),
            pltpu.VMEM((N_DEV, DC, 2 * HB), BF),
            pltpu.VMEM((N_DEV, DC, 2 * HB), BF),
            pltpu.VMEM((BS, 2 * HB), F32),
            pltpu.VMEM((BS, HB), BF),
            pltpu.VMEM((BS, HB), BF),
            pltpu.VMEM((BS, HB), BF),
            pltpu.VMEM((BS, RB), BF),
            pltpu.VMEM((BS, Dr), BF),
            pltpu.VMEM((BS, HB), BF),
            pltpu.VMEM((BS, D), BF),
            pltpu.VMEM((D, D), BF),
            pltpu.SemaphoreType.DMA((N_BUF, N_PEER)),
            pltpu.SemaphoreType.DMA((N_BUF, N_PEER)),
            pltpu.SemaphoreType.DMA((1,)),
        ],
        compiler_params=pltpu.CompilerParams(collective_id=0),
    )(x2, Wdkv, Wuk, Wuv, wq_m, wqr_m, Wkr, Wo)
    return out3
